# baseline (speedup 1.0000x reference)
import functools

import jax
import jax.numpy as jnp
from jax import lax
from jax.experimental import pallas as pl
from jax.experimental.pallas import tpu as pltpu
from jax.experimental.pallas import tpu_sc as plsc

_B, _T, _IN = 8, 512, 80
_H, _K, _D = 512, 1024, 512


def _mm(a, b):
    return lax.dot_general(a, b, (((1,), (0,)), ((), ())),
                           preferred_element_type=jnp.float32)


def _imcol(x):
    zrow = jnp.zeros((1, x.shape[1]), jnp.float32)
    xm = jnp.concatenate([zrow, x[:-1]], axis=0)
    xp = jnp.concatenate([x[1:], zrow], axis=0)
    return jnp.concatenate([xm, x, xp], axis=1)


def _enc_body(x_ref, w1_ref, b1_ref, w2_ref, b2_ref, ct_ref, idx_ref):
    x = x_ref[0]
    z1 = jnp.maximum(_mm(_imcol(x), w1_ref[...]) + b1_ref[...], 0.0)
    z2 = _mm(_imcol(z1), w2_ref[...]) + b2_ref[...]
    ct = ct_ref[...]
    dot = _mm(z2, ct)
    csq = jnp.sum(ct * ct, axis=0, keepdims=True)
    d = csq - 2.0 * dot
    m = jnp.min(d, axis=1, keepdims=True)
    cols = lax.broadcasted_iota(jnp.int32, d.shape, 1)
    idx = jnp.min(jnp.where(d == m, cols, _K), axis=1).astype(jnp.int32)
    idx_ref[...] = idx.reshape(1, 1, _T)


def _dec_body(q_ref, w3_ref, b3_ref, w4_ref, b4_ref, out_ref):
    q = q_ref[0]
    z = jnp.maximum(_mm(_imcol(q), w3_ref[...]) + b3_ref[...], 0.0)
    r = _mm(_imcol(z), w4_ref[...]) + b4_ref[...]
    out_ref[...] = r.reshape(1, _T, _IN)


_TC_PARAMS = pltpu.CompilerParams(dimension_semantics=("parallel",))


def _encode(mels, w1k, b1r, w2k, b2r, ct):
    return pl.pallas_call(
        _enc_body,
        grid=(_B,),
        in_specs=[
            pl.BlockSpec((1, _T, _IN), lambda b: (b, 0, 0)),
            pl.BlockSpec((3 * _IN, _H), lambda b: (0, 0)),
            pl.BlockSpec((1, _H), lambda b: (0, 0)),
            pl.BlockSpec((3 * _H, _D), lambda b: (0, 0)),
            pl.BlockSpec((1, _D), lambda b: (0, 0)),
            pl.BlockSpec((_D, _K), lambda b: (0, 0)),
        ],
        out_specs=pl.BlockSpec((1, 1, _T), lambda b: (b, 0, 0)),
        out_shape=jax.ShapeDtypeStruct((_B, 1, _T), jnp.int32),
        compiler_params=_TC_PARAMS,
    )(mels, w1k, b1r, w2k, b2r, ct)


def _decode(quant, w3k, b3r, w4k, b4r):
    return pl.pallas_call(
        _dec_body,
        grid=(_B,),
        in_specs=[
            pl.BlockSpec((1, _T, _D), lambda b: (b, 0, 0)),
            pl.BlockSpec((3 * _D, _H), lambda b: (0, 0)),
            pl.BlockSpec((1, _H), lambda b: (0, 0)),
            pl.BlockSpec((3 * _H, _IN), lambda b: (0, 0)),
            pl.BlockSpec((1, _IN), lambda b: (0, 0)),
        ],
        out_specs=pl.BlockSpec((1, _T, _IN), lambda b: (b, 0, 0)),
        out_shape=jax.ShapeDtypeStruct((_B, _T, _IN), jnp.float32),
        compiler_params=_TC_PARAMS,
    )(quant, w3k, b3r, w4k, b4r)


_NC, _NS = 2, 16
_NW = _NC * _NS
_BPW = (_B * _T) // _NW


_NBUF = 4
_CH = _BPW // _NBUF


def _sc_gather(codebook, idx_flat):
    mesh = plsc.VectorSubcoreMesh(core_axis_name="c", subcore_axis_name="s")

    @functools.partial(
        pl.kernel,
        mesh=mesh,
        out_type=jax.ShapeDtypeStruct((_B * _T, _D), jnp.float32),
        scratch_types=(
            [pltpu.VMEM((_BPW,), jnp.int32)]
            + [pltpu.VMEM((_CH, _D), jnp.float32) for _ in range(_NBUF)]
            + [pltpu.SemaphoreType.DMA for _ in range(2 * _NBUF)]
        ),
    )
    def k(table_hbm, idx_hbm, out_hbm, idx_v, *rest):
        bufs = rest[:_NBUF]
        gsems = rest[_NBUF:2 * _NBUF]
        wsems = rest[2 * _NBUF:]
        wid = lax.axis_index("s") * _NC + lax.axis_index("c")
        base = wid * _BPW
        pltpu.sync_copy(idx_hbm.at[pl.ds(base, _BPW)], idx_v)
        gathers = [
            pltpu.async_copy(table_hbm.at[idx_v.at[pl.ds(c * _CH, _CH)]],
                             bufs[c], gsems[c])
            for c in range(_NBUF)
        ]
        writes = []
        for c in range(_NBUF):
            gathers[c].wait()
            writes.append(pltpu.async_copy(
                bufs[c], out_hbm.at[pl.ds(base + c * _CH, _CH)], wsems[c]))
        for w in writes:
            w.wait()

    return k(codebook, idx_flat)


def kernel(mels, W1, b1, W2, b2, codebook, W3, b3, W4, b4):
    w1k = jnp.transpose(W1, (2, 1, 0)).reshape(3 * _IN, _H)
    w2k = jnp.transpose(W2, (2, 1, 0)).reshape(3 * _H, _D)
    w3k = jnp.transpose(W3, (2, 1, 0)).reshape(3 * _D, _H)
    w4k = jnp.transpose(W4, (2, 1, 0)).reshape(3 * _H, _IN)
    ct = jnp.transpose(codebook)

    idx3 = _encode(mels, w1k, b1.reshape(1, _H), w2k, b2.reshape(1, _D), ct)
    idx = idx3.reshape(_B, _T)
    quant = _sc_gather(codebook, idx.reshape(-1))
    recon = _decode(quant.reshape(_B, _T, _D), w3k, b3.reshape(1, _H),
                    w4k, b4.reshape(1, _IN))
    return (recon, idx)


# speedup vs baseline: 1.9992x; 1.9992x over previous
import jax
import jax.numpy as jnp
from jax import lax
from jax.experimental import pallas as pl
from jax.experimental.pallas import tpu as pltpu

_B, _T, _IN = 8, 512, 80
_H, _K, _D = 512, 1024, 512


def _mm(a, b):
    return lax.dot_general(a, b, (((1,), (0,)), ((), ())),
                           preferred_element_type=jnp.float32)


def _imcol(x):
    zrow = jnp.zeros((1, x.shape[1]), jnp.float32)
    xm = jnp.concatenate([zrow, x[:-1]], axis=0)
    xp = jnp.concatenate([x[1:], zrow], axis=0)
    return jnp.concatenate([xm, x, xp], axis=1)


def _body(x_ref, w1_ref, b1_ref, w2_ref, b2_ref, ct_ref, cb_ref,
          w3_ref, b3_ref, w4_ref, b4_ref, idx_ref, out_ref):
    x = x_ref[0]
    z1 = jnp.maximum(_mm(_imcol(x), w1_ref[...]) + b1_ref[...], 0.0)
    z2 = _mm(_imcol(z1), w2_ref[...]) + b2_ref[...]
    ct = ct_ref[...]
    dot = _mm(z2, ct)
    csq = jnp.sum(ct * ct, axis=0, keepdims=True)
    d = csq - 2.0 * dot
    m = jnp.min(d, axis=1, keepdims=True)
    cols = lax.broadcasted_iota(jnp.int32, d.shape, 1)
    idx = jnp.min(jnp.where(d == m, cols, _K), axis=1).astype(jnp.int32)
    onehot = (cols == idx[:, None]).astype(jnp.float32)
    quant = _mm(onehot, cb_ref[...])
    z3 = jnp.maximum(_mm(_imcol(quant), w3_ref[...]) + b3_ref[...], 0.0)
    r = _mm(_imcol(z3), w4_ref[...]) + b4_ref[...]
    idx_ref[...] = idx.reshape(1, 1, _T)
    out_ref[...] = r.reshape(1, _T, _IN)


_TC_PARAMS = pltpu.CompilerParams(dimension_semantics=("arbitrary",))


def kernel(mels, W1, b1, W2, b2, codebook, W3, b3, W4, b4):
    w1k = jnp.transpose(W1, (2, 1, 0)).reshape(3 * _IN, _H)
    w2k = jnp.transpose(W2, (2, 1, 0)).reshape(3 * _H, _D)
    w3k = jnp.transpose(W3, (2, 1, 0)).reshape(3 * _D, _H)
    w4k = jnp.transpose(W4, (2, 1, 0)).reshape(3 * _H, _IN)
    ct = jnp.transpose(codebook)

    idx3, recon = pl.pallas_call(
        _body,
        grid=(_B,),
        in_specs=[
            pl.BlockSpec((1, _T, _IN), lambda b: (b, 0, 0)),
            pl.BlockSpec((3 * _IN, _H), lambda b: (0, 0)),
            pl.BlockSpec((1, _H), lambda b: (0, 0)),
            pl.BlockSpec((3 * _H, _D), lambda b: (0, 0)),
            pl.BlockSpec((1, _D), lambda b: (0, 0)),
            pl.BlockSpec((_D, _K), lambda b: (0, 0)),
            pl.BlockSpec((_K, _D), lambda b: (0, 0)),
            pl.BlockSpec((3 * _D, _H), lambda b: (0, 0)),
            pl.BlockSpec((1, _H), lambda b: (0, 0)),
            pl.BlockSpec((3 * _H, _IN), lambda b: (0, 0)),
            pl.BlockSpec((1, _IN), lambda b: (0, 0)),
        ],
        out_specs=[
            pl.BlockSpec((1, 1, _T), lambda b: (b, 0, 0)),
            pl.BlockSpec((1, _T, _IN), lambda b: (b, 0, 0)),
        ],
        out_shape=[
            jax.ShapeDtypeStruct((_B, 1, _T), jnp.int32),
            jax.ShapeDtypeStruct((_B, _T, _IN), jnp.float32),
        ],
        compiler_params=_TC_PARAMS,
    )(mels, w1k, b1.reshape(1, _H), w2k, b2.reshape(1, _D), ct, codebook,
      w3k, b3.reshape(1, _H), w4k, b4.reshape(1, _IN))

    return (recon, idx3.reshape(_B, _T))


# speedup vs baseline: 2.0250x; 1.0129x over previous
import jax
import jax.numpy as jnp
from jax import lax
from jax.experimental import pallas as pl
from jax.experimental.pallas import tpu as pltpu

_B, _T, _IN = 8, 512, 80
_H, _K, _D = 512, 1024, 512


def _mm(a, b):
    return lax.dot_general(a, b, (((1,), (0,)), ((), ())),
                           preferred_element_type=jnp.float32)


def _imcol(x):
    zrow = jnp.zeros((1, x.shape[1]), x.dtype)
    xm = jnp.concatenate([zrow, x[:-1]], axis=0)
    xp = jnp.concatenate([x[1:], zrow], axis=0)
    return jnp.concatenate([xm, x, xp], axis=1)


def _body(x_ref, w1_ref, b1_ref, w2_ref, b2_ref, ct_ref, cb_ref,
          w3_ref, b3_ref, w4_ref, b4_ref, idx_ref, out_ref):
    x = x_ref[0]
    z1 = jnp.maximum(_mm(_imcol(x), w1_ref[...]) + b1_ref[...], 0.0)
    z2 = _mm(_imcol(z1), w2_ref[...]) + b2_ref[...]
    ct = ct_ref[...]
    dot = _mm(z2, ct)
    csq = jnp.sum(ct * ct, axis=0, keepdims=True)
    d = csq - 2.0 * dot
    m = jnp.min(d, axis=1, keepdims=True)
    cols = lax.broadcasted_iota(jnp.int32, d.shape, 1)
    idx = jnp.min(jnp.where(d == m, cols, _K), axis=1).astype(jnp.int32)
    onehot = (cols == idx[:, None]).astype(jnp.bfloat16)
    quant = _mm(onehot, cb_ref[...])
    z3 = jnp.maximum(
        _mm(_imcol(quant.astype(jnp.bfloat16)), w3_ref[...]) + b3_ref[...],
        0.0)
    r = _mm(_imcol(z3.astype(jnp.bfloat16)), w4_ref[...]) + b4_ref[...]
    idx_ref[...] = idx.reshape(1, 1, _T)
    out_ref[...] = r.reshape(1, _T, _IN)


_TC_PARAMS = pltpu.CompilerParams(dimension_semantics=("arbitrary",))


def kernel(mels, W1, b1, W2, b2, codebook, W3, b3, W4, b4):
    w1k = jnp.transpose(W1, (2, 1, 0)).reshape(3 * _IN, _H)
    w2k = jnp.transpose(W2, (2, 1, 0)).reshape(3 * _H, _D)
    w3k = jnp.transpose(W3, (2, 1, 0)).reshape(3 * _D, _H).astype(jnp.bfloat16)
    w4k = jnp.transpose(W4, (2, 1, 0)).reshape(3 * _H, _IN).astype(jnp.bfloat16)
    ct = jnp.transpose(codebook)
    cbb = codebook.astype(jnp.bfloat16)

    idx3, recon = pl.pallas_call(
        _body,
        grid=(_B,),
        in_specs=[
            pl.BlockSpec((1, _T, _IN), lambda b: (b, 0, 0)),
            pl.BlockSpec((3 * _IN, _H), lambda b: (0, 0)),
            pl.BlockSpec((1, _H), lambda b: (0, 0)),
            pl.BlockSpec((3 * _H, _D), lambda b: (0, 0)),
            pl.BlockSpec((1, _D), lambda b: (0, 0)),
            pl.BlockSpec((_D, _K), lambda b: (0, 0)),
            pl.BlockSpec((_K, _D), lambda b: (0, 0)),
            pl.BlockSpec((3 * _D, _H), lambda b: (0, 0)),
            pl.BlockSpec((1, _H), lambda b: (0, 0)),
            pl.BlockSpec((3 * _H, _IN), lambda b: (0, 0)),
            pl.BlockSpec((1, _IN), lambda b: (0, 0)),
        ],
        out_specs=[
            pl.BlockSpec((1, 1, _T), lambda b: (b, 0, 0)),
            pl.BlockSpec((1, _T, _IN), lambda b: (b, 0, 0)),
        ],
        out_shape=[
            jax.ShapeDtypeStruct((_B, 1, _T), jnp.int32),
            jax.ShapeDtypeStruct((_B, _T, _IN), jnp.float32),
        ],
        compiler_params=_TC_PARAMS,
    )(mels, w1k, b1.reshape(1, _H), w2k, b2.reshape(1, _D), ct, cbb,
      w3k, b3.reshape(1, _H), w4k, b4.reshape(1, _IN))

    return (recon, idx3.reshape(_B, _T))


# speedup vs baseline: 2.1200x; 1.0469x over previous
import jax
import jax.numpy as jnp
from jax import lax
from jax.experimental import pallas as pl
from jax.experimental.pallas import tpu as pltpu

_B, _T, _IN = 8, 512, 80
_H, _K, _D = 512, 1024, 512


def _mm(a, b):
    return lax.dot_general(a, b, (((1,), (0,)), ((), ())),
                           preferred_element_type=jnp.float32)


_BS = 2


def _imcol(x):
    zrow = jnp.zeros((1, x.shape[1]), x.dtype)
    m_parts, p_parts = [], []
    for s in range(_BS):
        lo = s * _T
        m_parts += [zrow, x[lo:lo + _T - 1]]
        p_parts += [x[lo + 1:lo + _T], zrow]
    xm = jnp.concatenate(m_parts, axis=0)
    xp = jnp.concatenate(p_parts, axis=0)
    return jnp.concatenate([xm, x, xp], axis=1)


def _body(x_ref, w1_ref, b1_ref, w2_ref, b2_ref, ct_ref, cb_ref,
          w3_ref, b3_ref, w4_ref, b4_ref, idx_ref, out_ref):
    x = x_ref[...].reshape(_BS * _T, _IN)
    z1 = jnp.maximum(_mm(_imcol(x), w1_ref[...]) + b1_ref[...], 0.0)
    z2 = _mm(_imcol(z1), w2_ref[...]) + b2_ref[...]
    ct = ct_ref[...]
    dot = _mm(z2, ct)
    csq = jnp.sum(ct * ct, axis=0, keepdims=True)
    d = csq - 2.0 * dot
    m = jnp.min(d, axis=1, keepdims=True)
    cols = lax.broadcasted_iota(jnp.int32, d.shape, 1)
    idx = jnp.min(jnp.where(d == m, cols, _K), axis=1).astype(jnp.int32)
    onehot = (cols == idx[:, None]).astype(jnp.bfloat16)
    quant = _mm(onehot, cb_ref[...])
    z3 = jnp.maximum(
        _mm(_imcol(quant.astype(jnp.bfloat16)), w3_ref[...]) + b3_ref[...],
        0.0)
    r = _mm(_imcol(z3.astype(jnp.bfloat16)), w4_ref[...]) + b4_ref[...]
    idx_ref[...] = idx.reshape(_BS, 1, _T)
    out_ref[...] = r.reshape(_BS, _T, _IN)


_TC_PARAMS = pltpu.CompilerParams(dimension_semantics=("arbitrary",))


def kernel(mels, W1, b1, W2, b2, codebook, W3, b3, W4, b4):
    w1k = jnp.transpose(W1, (2, 1, 0)).reshape(3 * _IN, _H)
    w2k = jnp.transpose(W2, (2, 1, 0)).reshape(3 * _H, _D)
    w3k = jnp.transpose(W3, (2, 1, 0)).reshape(3 * _D, _H).astype(jnp.bfloat16)
    w4k = jnp.transpose(W4, (2, 1, 0)).reshape(3 * _H, _IN).astype(jnp.bfloat16)
    ct = jnp.transpose(codebook)
    cbb = codebook.astype(jnp.bfloat16)

    idx3, recon = pl.pallas_call(
        _body,
        grid=(_B // _BS,),
        in_specs=[
            pl.BlockSpec((_BS, _T, _IN), lambda b: (b, 0, 0)),
            pl.BlockSpec((3 * _IN, _H), lambda b: (0, 0)),
            pl.BlockSpec((1, _H), lambda b: (0, 0)),
            pl.BlockSpec((3 * _H, _D), lambda b: (0, 0)),
            pl.BlockSpec((1, _D), lambda b: (0, 0)),
            pl.BlockSpec((_D, _K), lambda b: (0, 0)),
            pl.BlockSpec((_K, _D), lambda b: (0, 0)),
            pl.BlockSpec((3 * _D, _H), lambda b: (0, 0)),
            pl.BlockSpec((1, _H), lambda b: (0, 0)),
            pl.BlockSpec((3 * _H, _IN), lambda b: (0, 0)),
            pl.BlockSpec((1, _IN), lambda b: (0, 0)),
        ],
        out_specs=[
            pl.BlockSpec((_BS, 1, _T), lambda b: (b, 0, 0)),
            pl.BlockSpec((_BS, _T, _IN), lambda b: (b, 0, 0)),
        ],
        out_shape=[
            jax.ShapeDtypeStruct((_B, 1, _T), jnp.int32),
            jax.ShapeDtypeStruct((_B, _T, _IN), jnp.float32),
        ],
        compiler_params=_TC_PARAMS,
    )(mels, w1k, b1.reshape(1, _H), w2k, b2.reshape(1, _D), ct, cbb,
      w3k, b3.reshape(1, _H), w4k, b4.reshape(1, _IN))

    return (recon, idx3.reshape(_B, _T))


# speedup vs baseline: 2.1373x; 1.0082x over previous
import jax
import jax.numpy as jnp
from jax import lax
from jax.experimental import pallas as pl
from jax.experimental.pallas import tpu as pltpu

_B, _T, _IN = 8, 512, 80
_H, _K, _D = 512, 1024, 512


def _mm(a, b):
    return lax.dot_general(a, b, (((1,), (0,)), ((), ())),
                           preferred_element_type=jnp.float32)


_BS = 4


def _imcol(x):
    zrow = jnp.zeros((1, x.shape[1]), x.dtype)
    m_parts, p_parts = [], []
    for s in range(_BS):
        lo = s * _T
        m_parts += [zrow, x[lo:lo + _T - 1]]
        p_parts += [x[lo + 1:lo + _T], zrow]
    xm = jnp.concatenate(m_parts, axis=0)
    xp = jnp.concatenate(p_parts, axis=0)
    return jnp.concatenate([xm, x, xp], axis=1)


def _body(x_ref, w1_ref, b1_ref, w2_ref, b2_ref, ct_ref, cb_ref,
          w3_ref, b3_ref, w4_ref, b4_ref, idx_ref, out_ref):
    x = x_ref[...].reshape(_BS * _T, _IN)
    z1 = jnp.maximum(_mm(_imcol(x), w1_ref[...]) + b1_ref[...], 0.0)
    z2 = _mm(_imcol(z1), w2_ref[...]) + b2_ref[...]
    ct = ct_ref[...]
    dot = _mm(z2, ct)
    csq = jnp.sum(ct * ct, axis=0, keepdims=True)
    d = csq - 2.0 * dot
    m = jnp.min(d, axis=1, keepdims=True)
    cols = lax.broadcasted_iota(jnp.int32, d.shape, 1)
    idx = jnp.min(jnp.where(d == m, cols, _K), axis=1).astype(jnp.int32)
    onehot = (cols == idx[:, None]).astype(jnp.bfloat16)
    quant = _mm(onehot, cb_ref[...])
    z3 = jnp.maximum(
        _mm(_imcol(quant.astype(jnp.bfloat16)), w3_ref[...]) + b3_ref[...],
        0.0)
    r = _mm(_imcol(z3.astype(jnp.bfloat16)), w4_ref[...]) + b4_ref[...]
    idx_ref[...] = idx.reshape(_BS, 1, _T)
    out_ref[...] = r.reshape(_BS, _T, _IN)


_TC_PARAMS = pltpu.CompilerParams(dimension_semantics=("arbitrary",))


def kernel(mels, W1, b1, W2, b2, codebook, W3, b3, W4, b4):
    w1k = jnp.transpose(W1, (2, 1, 0)).reshape(3 * _IN, _H)
    w2k = jnp.transpose(W2, (2, 1, 0)).reshape(3 * _H, _D)
    w3k = jnp.transpose(W3, (2, 1, 0)).reshape(3 * _D, _H).astype(jnp.bfloat16)
    w4k = jnp.transpose(W4, (2, 1, 0)).reshape(3 * _H, _IN).astype(jnp.bfloat16)
    ct = jnp.transpose(codebook)
    cbb = codebook.astype(jnp.bfloat16)

    idx3, recon = pl.pallas_call(
        _body,
        grid=(_B // _BS,),
        in_specs=[
            pl.BlockSpec((_BS, _T, _IN), lambda b: (b, 0, 0)),
            pl.BlockSpec((3 * _IN, _H), lambda b: (0, 0)),
            pl.BlockSpec((1, _H), lambda b: (0, 0)),
            pl.BlockSpec((3 * _H, _D), lambda b: (0, 0)),
            pl.BlockSpec((1, _D), lambda b: (0, 0)),
            pl.BlockSpec((_D, _K), lambda b: (0, 0)),
            pl.BlockSpec((_K, _D), lambda b: (0, 0)),
            pl.BlockSpec((3 * _D, _H), lambda b: (0, 0)),
            pl.BlockSpec((1, _H), lambda b: (0, 0)),
            pl.BlockSpec((3 * _H, _IN), lambda b: (0, 0)),
            pl.BlockSpec((1, _IN), lambda b: (0, 0)),
        ],
        out_specs=[
            pl.BlockSpec((_BS, 1, _T), lambda b: (b, 0, 0)),
            pl.BlockSpec((_BS, _T, _IN), lambda b: (b, 0, 0)),
        ],
        out_shape=[
            jax.ShapeDtypeStruct((_B, 1, _T), jnp.int32),
            jax.ShapeDtypeStruct((_B, _T, _IN), jnp.float32),
        ],
        compiler_params=_TC_PARAMS,
    )(mels, w1k, b1.reshape(1, _H), w2k, b2.reshape(1, _D), ct, cbb,
      w3k, b3.reshape(1, _H), w4k, b4.reshape(1, _IN))

    return (recon, idx3.reshape(_B, _T))
